# Initial kernel scaffold; baseline (speedup 1.0000x reference)
import jax
import jax.numpy as jnp
from jax.experimental import pallas as pl


def kernel(nodes, edges, senders, receivers, globals_):
    raise NotImplementedError("write your pallas kernel here")


# speedup vs baseline: 9.9784x; 9.9784x over previous
import functools

import jax
import jax.numpy as jnp
from jax import lax
from jax.experimental import pallas as pl
from jax.experimental.pallas import tpu as pltpu
from jax.experimental.pallas import tpu_sc as plsc

STEP = 0.001
NC = 2
NS = 16
LANES = 16


def _sc_force_kernel(N, E, NPAD, CH, SUB):
    NW = NC * NS
    e_per_w = E // NW
    n_chunks = e_per_w // CH
    n_sub = CH // SUB
    rpt = NPAD // NS

    mesh = plsc.VectorSubcoreMesh(
        core_axis_name="c", subcore_axis_name="s",
        num_cores=NC, num_subcores=NS)

    @functools.partial(
        pl.kernel,
        out_type=(
            jax.ShapeDtypeStruct((E, 2), jnp.float32),
            jax.ShapeDtypeStruct((2 * NC, NPAD), jnp.float32),
        ),
        mesh=mesh,
        scratch_types=[
            pltpu.VMEM((CH,), jnp.int32),
            pltpu.VMEM((CH,), jnp.int32),
            pltpu.VMEM((CH // SUB, SUB), jnp.int32),
            pltpu.VMEM((CH,), jnp.float32),
            pltpu.VMEM((CH,), jnp.float32),
            pltpu.VMEM((CH,), jnp.float32),
            pltpu.VMEM((CH,), jnp.float32),
            pltpu.VMEM((CH,), jnp.float32),
            pltpu.VMEM((CH,), jnp.float32),
            pltpu.VMEM((CH,), jnp.float32),
            pltpu.VMEM((CH,), jnp.float32),
            pltpu.VMEM((CH, 2), jnp.float32),
            pltpu.VMEM((NPAD // NS,), jnp.float32),
            pltpu.VMEM_SHARED((NPAD,), jnp.float32),
            pltpu.VMEM_SHARED((NPAD,), jnp.float32),
            pltpu.VMEM_SHARED((NPAD,), jnp.float32),
            pltpu.VMEM_SHARED((NPAD,), jnp.float32),
        ],
        compiler_params=pltpu.CompilerParams(
            needs_layout_passes=False, use_tc_tiling_on_sc=False),
    )
    def sc_kernel(posx_hbm, posy_hbm, send_hbm, recv_hbm,
                  k_hbm, xr_hbm, zinit_hbm, force_out, partial_out,
                  sidx_v, ridx_v, ridx2_v, k_v, xr_v, sx_v, sy_v, rx_v, ry_v,
                  fx_v, fy_v, force_v, stage_v,
                  tabx_sh, taby_sh, accx_sh, accy_sh):
        c = lax.axis_index("c")
        s = lax.axis_index("s")
        w = c * NS + s

        r0 = s * rpt
        pltpu.sync_copy(posx_hbm.at[pl.ds(r0, rpt)], stage_v)
        pltpu.sync_copy(stage_v, tabx_sh.at[pl.ds(r0, rpt)])
        pltpu.sync_copy(posy_hbm.at[pl.ds(r0, rpt)], stage_v)
        pltpu.sync_copy(stage_v, taby_sh.at[pl.ds(r0, rpt)])
        pltpu.sync_copy(zinit_hbm.at[pl.ds(r0, rpt)], stage_v)
        pltpu.sync_copy(stage_v, accx_sh.at[pl.ds(r0, rpt)])
        pltpu.sync_copy(stage_v, accy_sh.at[pl.ds(r0, rpt)])
        plsc.subcore_barrier()

        base_e = w * e_per_w
        z16 = jnp.zeros((LANES,), jnp.int32)
        o16 = jnp.ones((LANES,), jnp.int32)

        def chunk_body(j, carry):
            off = base_e + j * CH
            pltpu.sync_copy(send_hbm.at[pl.ds(off, CH)], sidx_v)
            pltpu.sync_copy(recv_hbm.at[pl.ds(off, CH)], ridx_v)
            for i in range(n_sub):
                pltpu.sync_copy(recv_hbm.at[pl.ds(off + i * SUB, SUB)],
                                ridx2_v.at[i])
            pltpu.sync_copy(k_hbm.at[pl.ds(off, CH)], k_v)
            pltpu.sync_copy(xr_hbm.at[pl.ds(off, CH)], xr_v)
            pltpu.sync_copy(tabx_sh.at[sidx_v], sx_v)
            pltpu.sync_copy(taby_sh.at[sidx_v], sy_v)
            pltpu.sync_copy(tabx_sh.at[ridx_v], rx_v)
            pltpu.sync_copy(taby_sh.at[ridx_v], ry_v)

            def edge_body(i, carry2):
                st = i * LANES
                kk = k_v[pl.ds(st, LANES)]
                xr = xr_v[pl.ds(st, LANES)]
                dx = rx_v[pl.ds(st, LANES)] - sx_v[pl.ds(st, LANES)]
                dy = ry_v[pl.ds(st, LANES)] - sy_v[pl.ds(st, LANES)]
                s2 = dx * dx + dy * dy
                ib = plsc.bitcast(s2, jnp.int32)
                y = plsc.bitcast(0x5F3759DF - (ib >> 1), jnp.float32)
                xh = 0.5 * s2
                y = y * (1.5 - xh * y * y)
                y = y * (1.5 - xh * y * y)
                y = y * (1.5 - xh * y * y)
                fmag = kk * (xr * y - 1.0)
                fx = fmag * dx
                fy = fmag * dy
                fx_v[pl.ds(st, LANES)] = fx
                fy_v[pl.ds(st, LANES)] = fy
                ids = lax.iota(jnp.int32, LANES) + st
                plsc.store_scatter(force_v, [ids, z16], fx)
                plsc.store_scatter(force_v, [ids, o16], fy)
                return carry2

            lax.fori_loop(0, CH // LANES, edge_body, 0)
            pltpu.sync_copy(force_v, force_out.at[pl.ds(off, CH)])
            for i in range(n_sub):
                pltpu.sync_copy(fx_v.at[pl.ds(i * SUB, SUB)],
                                accx_sh.at[ridx2_v.at[i]], add=True)
                pltpu.sync_copy(fy_v.at[pl.ds(i * SUB, SUB)],
                                accy_sh.at[ridx2_v.at[i]], add=True)
            return carry

        lax.fori_loop(0, n_chunks, chunk_body, 0)
        plsc.subcore_barrier()
        pltpu.sync_copy(accx_sh.at[pl.ds(r0, rpt)], stage_v)
        pltpu.sync_copy(stage_v, partial_out.at[2 * c, pl.ds(r0, rpt)])
        pltpu.sync_copy(accy_sh.at[pl.ds(r0, rpt)], stage_v)
        pltpu.sync_copy(stage_v, partial_out.at[2 * c + 1, pl.ds(r0, rpt)])

    return sc_kernel


def _euler_tc_kernel(vel_ref, mfree_ref, g_ref, psum_ref, out_ref):
    px = psum_ref[0] + psum_ref[2]
    py = psum_ref[1] + psum_ref[3]
    m = mfree_ref[...]
    out_ref[0] = vel_ref[0] + (px + g_ref[0, 0]) * m * STEP
    out_ref[1] = vel_ref[1] + (py + g_ref[0, 1]) * m * STEP


def kernel(nodes, edges, senders, receivers, globals_):
    N = nodes.shape[0]
    E = senders.shape[0]
    NW = NC * NS
    CH = 2000
    SUB = 80
    assert E % (NW * CH) == 0
    NPAD = ((N + 8 * NS - 1) // (8 * NS)) * (8 * NS)

    posx = jnp.pad(nodes[:, 0], (0, NPAD - N))
    posy = jnp.pad(nodes[:, 1], (0, NPAD - N))
    k_spring = edges[:, 0]
    x_rest = edges[:, 1]
    zinit = jnp.zeros((NPAD,), jnp.float32)

    force, partial = _sc_force_kernel(N, E, NPAD, CH, SUB)(
        posx, posy, senders, receivers, k_spring, x_rest, zinit)

    rows = NPAD // 128
    vel_p = jnp.pad(nodes[:, 2:4], ((0, NPAD - N), (0, 0)))
    mfree_p = jnp.pad(1.0 - nodes[:, 4], (0, NPAD - N))

    updated = pl.pallas_call(
        _euler_tc_kernel,
        out_shape=jax.ShapeDtypeStruct((2, rows, 128), jnp.float32),
    )(
        vel_p.T.reshape(2, rows, 128),
        mfree_p.reshape(rows, 128),
        globals_,
        partial.reshape(2 * NC, rows, 128),
    )
    updated_velocities = updated.reshape(2, NPAD).T[:N]
    return (force, updated_velocities)
